# baseline (speedup 1.0000x reference)
import jax
import jax.numpy as jnp
from jax import lax
from jax.experimental import pallas as pl
from jax.experimental.pallas import tpu as pltpu
from jax.experimental.pallas import tpu_sc as plsc

B = 1024
S = 200
H = 128
SP = 208
NC = 2
NS = 16
NW = NC * NS
ROWS_PER_W = B // NW
LANES = 16
KCH = H // LANES
NQ = SP // LANES


def _body(ids_h, cidx_h, pos_h, typ_h, ten_h, word_h, out_h,
          pos_v, combo_v, typ_v, ids_v, cidx_v, acc_v,
          gsem, wsem):
    c = lax.axis_index("c")
    s = lax.axis_index("s")
    wid = s * NC + c

    pltpu.sync_copy(ids_h.at[pl.ds(wid * ROWS_PER_W * SP, ROWS_PER_W * SP)],
                    ids_v)
    pltpu.sync_copy(cidx_h.at[pl.ds(wid * ROWS_PER_W * SP, ROWS_PER_W * SP)],
                    cidx_v)
    pltpu.sync_copy(pos_h, pos_v)
    pltpu.sync_copy(typ_h, typ_v)
    pltpu.sync_copy(ten_h, acc_v.at[0, pl.ds(0, 104)])

    def build(t, carry):
        for half in range(2):
            for k in range(KCH):
                sl = pl.ds(k * LANES, LANES)
                combo_v[half * 100 + t, sl] = (
                    acc_v[0, t, sl]
                    + typ_v[pl.ds(half * H + k * LANES, LANES)])
        return carry
    lax.fori_loop(0, 100, build, 0)

    def issue_gather(r, slot):
        pltpu.async_copy(
            word_h.at[ids_v.at[pl.ds(r * SP, 104)]],
            acc_v.at[slot, pl.ds(0, 104)], gsem.at[slot])
        pltpu.async_copy(
            word_h.at[ids_v.at[pl.ds(r * SP + 104, 104)]],
            acc_v.at[slot, pl.ds(104, 104)], gsem.at[slot])

    def wait_gather(r, slot):
        pltpu.make_async_copy(
            word_h.at[ids_v.at[pl.ds(r * SP, 104)]],
            acc_v.at[slot, pl.ds(0, 104)], gsem.at[slot]).wait()
        pltpu.make_async_copy(
            word_h.at[ids_v.at[pl.ds(r * SP + 104, 104)]],
            acc_v.at[slot, pl.ds(104, 104)], gsem.at[slot]).wait()

    def issue_wb(r, slot):
        b = wid * ROWS_PER_W + r
        pltpu.async_copy(
            acc_v.at[slot, pl.ds(0, S)], out_h.at[b], wsem.at[slot])

    def wait_wb(r, slot):
        b = wid * ROWS_PER_W + r
        pltpu.make_async_copy(
            acc_v.at[slot, pl.ds(0, S)], out_h.at[b], wsem.at[slot]).wait()

    issue_gather(0, 0)

    def row(r, carry):
        sl = lax.rem(r, 2)
        nx = 1 - sl

        @pl.when(r >= 1)
        def _():
            wait_wb(r - 1, nx)

        @pl.when(r < ROWS_PER_W - 1)
        def _():
            issue_gather(r + 1, nx)

        wait_gather(r, sl)

        def group(q, inner):
            t0 = q * LANES
            chunk = cidx_v[pl.ds(r * SP + t0, LANES)]
            for i in range(LANES):
                ct = chunk[i]
                t = t0 + i
                for k in range(KCH):
                    ssl = pl.ds(k * LANES, LANES)
                    acc_v[sl, t, ssl] = (acc_v[sl, t, ssl]
                                         + pos_v[t, ssl]
                                         + combo_v[ct, ssl])
            return inner
        lax.fori_loop(0, NQ, group, 0)

        issue_wb(r, sl)
        return carry
    lax.fori_loop(0, ROWS_PER_W, row, 0)

    wait_wb(ROWS_PER_W - 1, 1)


@jax.jit
def _run(ids, cidx, pos, typ, ten, word):
    mesh = plsc.VectorSubcoreMesh(core_axis_name="c", subcore_axis_name="s")
    return pl.kernel(
        _body,
        out_type=jax.ShapeDtypeStruct((B, S, H), jnp.float32),
        mesh=mesh,
        scratch_types=[
            pltpu.VMEM((SP, H), jnp.float32),
            pltpu.VMEM((S, H), jnp.float32),
            pltpu.VMEM((2 * H,), jnp.float32),
            pltpu.VMEM((ROWS_PER_W * SP,), jnp.int32),
            pltpu.VMEM((ROWS_PER_W * SP,), jnp.int32),
            pltpu.VMEM((2, SP, H), jnp.float32),
            pltpu.SemaphoreType.DMA((2,)),
            pltpu.SemaphoreType.DMA((2,)),
        ],
    )(ids, cidx, pos, typ, ten, word)


def kernel(input_ids, token_type_ids, tenant_ids, W_word, W_pos, W_type, W_tenant):
    ids = input_ids.astype(jnp.int32)
    cidx = (token_type_ids.astype(jnp.int32) * 100
            + tenant_ids.astype(jnp.int32))
    ids_p = jnp.zeros((B, SP), jnp.int32).at[:, :S].set(ids)
    cidx_p = jnp.zeros((B, SP), jnp.int32).at[:, :S].set(cidx)
    pos = jnp.pad(W_pos[:S], ((0, SP - S), (0, 0)))
    ten = jnp.pad(W_tenant, ((0, 104 - W_tenant.shape[0]), (0, 0)))
    return _run(ids_p.reshape(-1), cidx_p.reshape(-1), pos,
                W_type.reshape(-1), ten, W_word)
